# Initial kernel scaffold; baseline (speedup 1.0000x reference)
import jax
import jax.numpy as jnp
from jax.experimental import pallas as pl


def kernel(x, duration, max_len):
    raise NotImplementedError("write your pallas kernel here")


# speedup vs baseline: 15.8041x; 15.8041x over previous
import functools

import jax
import jax.numpy as jnp
from jax import lax
from jax.experimental import pallas as pl
from jax.experimental.pallas import tpu as pltpu
from jax.experimental.pallas import tpu_sc as plsc

_B, _T, _D = 16, 512, 256
_MAX_LEN = 2048
_ZROW = _B * _T
_NW = 32
_ROWS_PER_W = _B * _MAX_LEN // _NW
_CHUNK = 128
_NCHUNK = _ROWS_PER_W // _CHUNK


def _idx_body(dur_ref, idx_ref, len_ref, cum_ref):
    dur = dur_ref[...]
    ii = lax.broadcasted_iota(jnp.int32, (_T, _T), 0)
    jj = lax.broadcasted_iota(jnp.int32, (_T, _T), 1)
    tri = (ii <= jj).astype(jnp.float32)
    cum = jnp.dot(dur.astype(jnp.float32), tri,
                  preferred_element_type=jnp.float32).astype(jnp.int32)
    total = cum[:, _T - 1]
    cum_ref[...] = cum
    pos = lax.broadcasted_iota(jnp.int32, (_MAX_LEN, 1), 0)

    def body(b, carry):
        cum_b = cum_ref[pl.ds(b, 1), :]
        cnt = jnp.sum((cum_b <= pos).astype(jnp.int32), axis=1)
        g = jnp.where(cnt >= _T, _ZROW, b * _T + cnt)
        idx_ref[pl.ds(b, 1), :] = g[None, :]
        return carry

    lax.fori_loop(0, _B, body, 0)
    len_ref[...] = jnp.broadcast_to(total[:, None], (_B, 128))


_idx_call = pl.pallas_call(
    _idx_body,
    out_shape=(
        jax.ShapeDtypeStruct((_B, _MAX_LEN), jnp.int32),
        jax.ShapeDtypeStruct((_B, 128), jnp.int32),
    ),
    scratch_shapes=[pltpu.VMEM((_B, _T), jnp.int32)],
)


_sc_mesh = plsc.VectorSubcoreMesh(core_axis_name="c", subcore_axis_name="s")


@functools.partial(
    pl.kernel,
    mesh=_sc_mesh,
    out_type=jax.ShapeDtypeStruct((_B * _MAX_LEN, _D), jnp.float32),
    scratch_types=[
        pltpu.VMEM((_NCHUNK, _CHUNK), jnp.int32),
        pltpu.VMEM((_CHUNK, _D), jnp.float32),
        pltpu.VMEM((_CHUNK, _D), jnp.float32),
        pltpu.SemaphoreType.DMA,
        pltpu.SemaphoreType.DMA,
    ],
)
def _gather_call(xext_hbm, idx_hbm, out_hbm, idx_v, buf0, buf1, sem0, sem1):
    wid = lax.axis_index("s") * 2 + lax.axis_index("c")
    base = wid * _ROWS_PER_W
    pltpu.sync_copy(idx_hbm.at[wid], idx_v)
    bufs = (buf0, buf1)
    sems = (sem0, sem1)
    cps = [None, None]
    cps[0] = pltpu.async_copy(xext_hbm.at[idx_v.at[0]], bufs[0], sems[0])
    for c in range(_NCHUNK):
        if c + 1 < _NCHUNK:
            k = (c + 1) % 2
            cps[k] = pltpu.async_copy(xext_hbm.at[idx_v.at[c + 1]], bufs[k], sems[k])
        cps[c % 2].wait()
        pltpu.sync_copy(bufs[c % 2], out_hbm.at[pl.ds(base + c * _CHUNK, _CHUNK)])


def kernel(x, duration, max_len):
    del max_len
    x_ext = jnp.concatenate(
        [x.reshape(_B * _T, _D), jnp.zeros((8, _D), jnp.float32)], axis=0)
    idx, mel = _idx_call(duration)
    out_flat = _gather_call(x_ext, idx.reshape(_NW, _NCHUNK, _CHUNK))
    return out_flat.reshape(_B, _MAX_LEN, _D), mel[:, 0]


# speedup vs baseline: 15.8503x; 1.0029x over previous
import functools

import jax
import jax.numpy as jnp
from jax import lax
from jax.experimental import pallas as pl
from jax.experimental.pallas import tpu as pltpu
from jax.experimental.pallas import tpu_sc as plsc

_B, _T, _D = 16, 512, 256
_MAX_LEN = 2048
_ZROW = _B * _T
_NW = 32
_ROWS_PER_W = _B * _MAX_LEN // _NW
_CHUNK = 64
_NCHUNK = _ROWS_PER_W // _CHUNK
_NBUF = 6


def _idx_body(dur_ref, idx_ref, len_ref, cum_ref):
    dur = dur_ref[...]
    ii = lax.broadcasted_iota(jnp.int32, (_T, _T), 0)
    jj = lax.broadcasted_iota(jnp.int32, (_T, _T), 1)
    tri = (ii <= jj).astype(jnp.float32)
    cum = jnp.dot(dur.astype(jnp.float32), tri,
                  preferred_element_type=jnp.float32).astype(jnp.int32)
    total = cum[:, _T - 1]
    cum_ref[...] = cum
    pos = lax.broadcasted_iota(jnp.int32, (_MAX_LEN, 1), 0)

    def body(b, carry):
        cum_b = cum_ref[pl.ds(b, 1), :]
        cnt = jnp.sum((cum_b <= pos).astype(jnp.int32), axis=1)
        g = jnp.where(cnt >= _T, _ZROW, b * _T + cnt)
        idx_ref[pl.ds(b, 1), :] = g[None, :]
        return carry

    lax.fori_loop(0, _B, body, 0)
    len_ref[...] = jnp.broadcast_to(total[:, None], (_B, 128))


_idx_call = pl.pallas_call(
    _idx_body,
    out_shape=(
        jax.ShapeDtypeStruct((_B, _MAX_LEN), jnp.int32),
        jax.ShapeDtypeStruct((_B, 128), jnp.int32),
    ),
    scratch_shapes=[pltpu.VMEM((_B, _T), jnp.int32)],
)


_sc_mesh = plsc.VectorSubcoreMesh(core_axis_name="c", subcore_axis_name="s")


@functools.partial(
    pl.kernel,
    mesh=_sc_mesh,
    out_type=jax.ShapeDtypeStruct((_B * _MAX_LEN, _D), jnp.float32),
    scratch_types=(
        [pltpu.VMEM((_NCHUNK, _CHUNK), jnp.int32)]
        + [pltpu.VMEM((_CHUNK, _D), jnp.float32) for _ in range(_NBUF)]
        + [pltpu.SemaphoreType.DMA for _ in range(2 * _NBUF)]
    ),
)
def _gather_call(xext_hbm, idx_hbm, out_hbm, idx_v, *bufs_sems):
    bufs = bufs_sems[:_NBUF]
    gsems = bufs_sems[_NBUF:2 * _NBUF]
    wsems = bufs_sems[2 * _NBUF:]
    wid = lax.axis_index("s") * 2 + lax.axis_index("c")
    base = wid * _ROWS_PER_W
    pltpu.sync_copy(idx_hbm.at[wid], idx_v)
    gcp = [None] * _NCHUNK
    for ch in range(_NBUF):
        gcp[ch] = pltpu.async_copy(xext_hbm.at[idx_v.at[ch]], bufs[ch],
                                   gsems[ch])
    for ch in range(_NCHUNK):
        slot = ch % _NBUF
        gcp[ch].wait()
        wcp = pltpu.async_copy(bufs[slot],
                               out_hbm.at[pl.ds(base + ch * _CHUNK, _CHUNK)],
                               wsems[slot])
        wcp.wait()
        nx = ch + _NBUF
        if nx < _NCHUNK:
            gcp[nx] = pltpu.async_copy(xext_hbm.at[idx_v.at[nx]], bufs[slot],
                                       gsems[slot])


def kernel(x, duration, max_len):
    del max_len
    x_ext = jnp.concatenate(
        [x.reshape(_B * _T, _D), jnp.zeros((8, _D), jnp.float32)], axis=0)
    idx, mel = _idx_call(duration)
    out_flat = _gather_call(x_ext, idx.reshape(_NW, _NCHUNK, _CHUNK))
    return out_flat.reshape(_B, _MAX_LEN, _D), mel[:, 0]


# speedup vs baseline: 16.1245x; 1.0173x over previous
import functools

import jax
import jax.numpy as jnp
from jax import lax
from jax.experimental import pallas as pl
from jax.experimental.pallas import tpu as pltpu
from jax.experimental.pallas import tpu_sc as plsc

_B, _T, _D = 16, 512, 256
_MAX_LEN = 2048
_ZROW = _B * _T
_NW = 32
_ROWS_PER_W = _B * _MAX_LEN // _NW
_CHUNK = 64
_NCHUNK = _ROWS_PER_W // _CHUNK
_NBUF = 6


def _idx_body(dur_ref, idx_ref, len_ref, cum_ref):
    dur = dur_ref[...]
    ii = lax.broadcasted_iota(jnp.int32, (_T, _T), 0)
    jj = lax.broadcasted_iota(jnp.int32, (_T, _T), 1)
    tri = (ii <= jj).astype(jnp.float32)
    cum = jnp.dot(dur.astype(jnp.float32), tri,
                  preferred_element_type=jnp.float32).astype(jnp.int32)
    total = cum[:, _T - 1]
    cum_ref[...] = cum
    pos = lax.broadcasted_iota(jnp.int32, (_MAX_LEN, 1), 0)

    def body(b, carry):
        cum_b = cum_ref[pl.ds(b, 1), :]
        cnt = jnp.sum((cum_b <= pos).astype(jnp.int32), axis=1)
        g = jnp.where(cnt >= _T, _ZROW, b * _T + cnt)
        idx_ref[pl.ds(b, 1), :] = g[None, :]
        return carry

    lax.fori_loop(0, _B, body, 0)
    len_ref[...] = jnp.broadcast_to(total[:, None], (_B, 128))


_idx_call = pl.pallas_call(
    _idx_body,
    out_shape=(
        jax.ShapeDtypeStruct((_B, _MAX_LEN), jnp.int32),
        jax.ShapeDtypeStruct((_B, 128), jnp.int32),
    ),
    scratch_shapes=[pltpu.VMEM((_B, _T), jnp.int32)],
)


_sc_mesh = plsc.VectorSubcoreMesh(core_axis_name="c", subcore_axis_name="s")


_CH_PER_PAIR = _B * _MAX_LEN // _CHUNK // 16
_K0 = 6


@functools.partial(
    pl.kernel,
    mesh=_sc_mesh,
    out_type=jax.ShapeDtypeStruct((_B * _MAX_LEN, _D), jnp.float32),
    scratch_types=(
        [pltpu.VMEM((_CH_PER_PAIR, _CHUNK), jnp.int32)]
        + [pltpu.VMEM((_CHUNK, _D), jnp.float32) for _ in range(_NBUF)]
        + [pltpu.SemaphoreType.DMA for _ in range(2 * _NBUF)]
    ),
)
def _gather_call(xext_hbm, idx_hbm, out_hbm, idx_v, *bufs_sems):
    bufs = bufs_sems[:_NBUF]
    gsems = bufs_sems[_NBUF:2 * _NBUF]
    wsems = bufs_sems[2 * _NBUF:]
    c = lax.axis_index("c")
    s = lax.axis_index("s")

    pltpu.sync_copy(idx_hbm.at[pl.ds(s * _CH_PER_PAIR, _CH_PER_PAIR)], idx_v)

    def pipe(chunk0, nchunks):
        gbase = s * _CH_PER_PAIR + chunk0
        gcp = [None] * nchunks
        for ch in range(min(_NBUF, nchunks)):
            gcp[ch] = pltpu.async_copy(xext_hbm.at[idx_v.at[chunk0 + ch]],
                                       bufs[ch], gsems[ch])
        for ch in range(nchunks):
            slot = ch % _NBUF
            gcp[ch].wait()
            wcp = pltpu.async_copy(
                bufs[slot],
                out_hbm.at[pl.ds((gbase + ch) * _CHUNK, _CHUNK)],
                wsems[slot])
            wcp.wait()
            nx = ch + _NBUF
            if nx < nchunks:
                gcp[nx] = pltpu.async_copy(xext_hbm.at[idx_v.at[chunk0 + nx]],
                                           bufs[slot], gsems[slot])

    @pl.when(c == 0)
    def _():
        pipe(0, _K0)

    @pl.when(c == 1)
    def _():
        pipe(_K0, _CH_PER_PAIR - _K0)


def kernel(x, duration, max_len):
    del max_len
    x_ext = jnp.concatenate(
        [x.reshape(_B * _T, _D), jnp.zeros((8, _D), jnp.float32)], axis=0)
    idx, mel = _idx_call(duration)
    out_flat = _gather_call(
        x_ext, idx.reshape(_B * _MAX_LEN // _CHUNK, _CHUNK))
    return out_flat.reshape(_B, _MAX_LEN, _D), mel[:, 0]
